# Initial kernel scaffold; baseline (speedup 1.0000x reference)
import functools

import jax
import jax.numpy as jnp
from jax import lax
from jax.experimental import pallas as pl
from jax.experimental.pallas import tpu as pltpu
from jax.experimental.pallas import tpu_sc as plsc

N = 50000
E = 800000
D = 64
H = 32

NC = 2
NS = 16
BG = 128
NSUB = 8
CHUNKS = 49
EDGES_PER_TILE = CHUNKS * NSUB * BG
E_PAD = NS * EDGES_PER_TILE
ROWS_PER_TILE = N // NS
ACC_ROWS = N + 8
BLOCKS = NS * CHUNKS

BM = 2000


def _tc_body(x_ref, wrel_ref, wroot_ref, b_ref,
             ylo_ref, yhi_ref, blo_ref, bhi_ref):
    x = x_ref[...]
    y = jnp.dot(x, wrel_ref[...], preferred_element_type=jnp.float32)
    base = jnp.dot(x, wroot_ref[...], preferred_element_type=jnp.float32)
    base = base + b_ref[...]
    ylo_ref[...] = y[:, :H]
    yhi_ref[...] = y[:, H:]
    blo_ref[...] = base[:, :H]
    bhi_ref[...] = base[:, H:]


def _tc_stage(features, W_rel, b_rel, W_root):
    grid = (N // BM,)
    half = jax.ShapeDtypeStruct((N, H), jnp.float32)
    return pl.pallas_call(
        _tc_body,
        grid=grid,
        in_specs=[
            pl.BlockSpec((BM, D), lambda i: (i, 0)),
            pl.BlockSpec((D, D), lambda i: (0, 0)),
            pl.BlockSpec((D, D), lambda i: (0, 0)),
            pl.BlockSpec((1, D), lambda i: (0, 0)),
        ],
        out_specs=[pl.BlockSpec((BM, H), lambda i: (i, 0))] * 4,
        out_shape=[half, half, half, half],
    )(features, W_rel, W_root, b_rel.reshape(1, D))


def _sc_body(ylo, yhi, blo, bhi, src_hbm, dst_hbm, out_lo, out_hi,
             acc, src_buf, dst_buf, rows, gsem, ssem):
    c = lax.axis_index("c")
    s = lax.axis_index("s")
    stripe = pl.ds(s * ROWS_PER_TILE, ROWS_PER_TILE)

    @pl.when(c == 0)
    def _():
        pltpu.sync_copy(blo.at[stripe], acc.at[stripe])

    @pl.when(c == 1)
    def _():
        pltpu.sync_copy(bhi.at[stripe], acc.at[stripe])

    plsc.subcore_barrier()

    def edge_loop(y_tab):
        def chunk(k, carry):
            b = s * CHUNKS + k
            pltpu.sync_copy(src_hbm.at[b], src_buf)
            pltpu.sync_copy(dst_hbm.at[b], dst_buf)
            gh = [pltpu.async_copy(y_tab.at[src_buf.at[j]], rows.at[j], gsem)
                  for j in range(NSUB)]
            for h in gh:
                h.wait()
            sh = [pltpu.async_copy(rows.at[j], acc.at[dst_buf.at[j]], ssem,
                                   add=True)
                  for j in range(NSUB)]
            for h in sh:
                h.wait()
            return carry
        lax.fori_loop(0, CHUNKS, chunk, 0)

    @pl.when(c == 0)
    def _():
        edge_loop(ylo)

    @pl.when(c == 1)
    def _():
        edge_loop(yhi)

    plsc.subcore_barrier()

    @pl.when(c == 0)
    def _():
        pltpu.sync_copy(acc.at[stripe], out_lo.at[stripe])

    @pl.when(c == 1)
    def _():
        pltpu.sync_copy(acc.at[stripe], out_hi.at[stripe])


def _sc_stage(ylo, yhi, blo, bhi, src_blocks, dst_blocks):
    half = jax.ShapeDtypeStruct((N, H), jnp.float32)
    mesh = plsc.VectorSubcoreMesh(core_axis_name="c", subcore_axis_name="s")
    run = pl.kernel(
        _sc_body,
        out_type=(half, half),
        mesh=mesh,
        scratch_types=[
            pltpu.VMEM_SHARED((ACC_ROWS, H), jnp.float32),
            pltpu.VMEM((NSUB, BG), jnp.int32),
            pltpu.VMEM((NSUB, BG), jnp.int32),
            pltpu.VMEM((NSUB, BG, H), jnp.float32),
            pltpu.SemaphoreType.DMA,
            pltpu.SemaphoreType.DMA,
        ],
    )
    return run(ylo, yhi, blo, bhi, src_blocks, dst_blocks)


@jax.jit
def kernel(features, edge_index, W_rel, b_rel, W_root):
    ylo, yhi, blo, bhi = _tc_stage(features, W_rel, b_rel, W_root)
    pad = E_PAD - E
    src = jnp.concatenate(
        [edge_index[0], jnp.zeros((pad,), jnp.int32)]).reshape(BLOCKS, NSUB, BG)
    dst = jnp.concatenate(
        [edge_index[1], jnp.full((pad,), N, jnp.int32)]).reshape(BLOCKS, NSUB, BG)
    out_lo, out_hi = _sc_stage(ylo, yhi, blo, bhi, src, dst)
    return jnp.concatenate([out_lo, out_hi], axis=1)


# speedup vs baseline: 7.0880x; 7.0880x over previous
import jax
import jax.numpy as jnp
from jax import lax
from jax.experimental import pallas as pl
from jax.experimental.pallas import tpu as pltpu
from jax.experimental.pallas import tpu_sc as plsc

N = 50000
NP = 50048
E = 800000
D = 64
H = 32

NC = 2
NS = 16
BG = 128
NSUB = 4
CHUNKS = 98
EDGES_PER_TILE = CHUNKS * NSUB * BG
E_PAD = NS * EDGES_PER_TILE
ROWS_PER_TILE = NP // NS
BLOCKS = NS * CHUNKS

BM = ROWS_PER_TILE


def _tc_body(x_ref, wrel_ref, wroot_ref, b_ref,
             ylo_ref, yhi_ref, blo_ref, bhi_ref):
    x = x_ref[...]
    y = jnp.dot(x, wrel_ref[...], preferred_element_type=jnp.float32)
    base = jnp.dot(x, wroot_ref[...], preferred_element_type=jnp.float32)
    base = base + b_ref[...]
    ylo_ref[...] = y[:, :H]
    yhi_ref[...] = y[:, H:]
    blo_ref[...] = base[:, :H]
    bhi_ref[...] = base[:, H:]


def _tc_stage(features, W_rel, b_rel, W_root):
    grid = (NP // BM,)
    half = jax.ShapeDtypeStruct((NP, H), jnp.float32)
    return pl.pallas_call(
        _tc_body,
        grid=grid,
        in_specs=[
            pl.BlockSpec((BM, D), lambda i: (i, 0)),
            pl.BlockSpec((D, D), lambda i: (0, 0)),
            pl.BlockSpec((D, D), lambda i: (0, 0)),
            pl.BlockSpec((1, D), lambda i: (0, 0)),
        ],
        out_specs=[pl.BlockSpec((BM, H), lambda i: (i, 0))] * 4,
        out_shape=[half, half, half, half],
    )(features, W_rel, W_root, b_rel.reshape(1, D))


def _sc_body(ylo, yhi, blo, bhi, src_hbm, dst_hbm, out_lo, out_hi,
             acc, src_buf, dst_buf, rows, gsem, ssem):
    c = lax.axis_index("c")
    s = lax.axis_index("s")
    stripe = pl.ds(s * ROWS_PER_TILE, ROWS_PER_TILE)

    @pl.when(c == 0)
    def _():
        pltpu.sync_copy(blo.at[stripe], acc.at[stripe])

    @pl.when(c == 1)
    def _():
        pltpu.sync_copy(bhi.at[stripe], acc.at[stripe])

    plsc.subcore_barrier()

    def edge_loop(y_tab):
        def chunk(k, carry):
            b = s * CHUNKS + k
            pltpu.sync_copy(src_hbm.at[b], src_buf)
            pltpu.sync_copy(dst_hbm.at[b], dst_buf)
            gh = [pltpu.async_copy(y_tab.at[src_buf.at[pl.ds(j * BG, BG)]],
                                   rows.at[j], gsem)
                  for j in range(NSUB)]
            for h in gh:
                h.wait()
            sh = [pltpu.async_copy(rows.at[j],
                                   acc.at[dst_buf.at[pl.ds(j * BG, BG)]], ssem,
                                   add=True)
                  for j in range(NSUB)]
            for h in sh:
                h.wait()
            return carry
        lax.fori_loop(0, CHUNKS, chunk, 0)

    @pl.when(c == 0)
    def _():
        edge_loop(ylo)

    @pl.when(c == 1)
    def _():
        edge_loop(yhi)

    plsc.subcore_barrier()

    @pl.when(c == 0)
    def _():
        pltpu.sync_copy(acc.at[stripe], out_lo.at[stripe])

    @pl.when(c == 1)
    def _():
        pltpu.sync_copy(acc.at[stripe], out_hi.at[stripe])


def _sc_stage(ylo, yhi, blo, bhi, src_blocks, dst_blocks):
    half = jax.ShapeDtypeStruct((NP, H), jnp.float32)
    mesh = plsc.VectorSubcoreMesh(core_axis_name="c", subcore_axis_name="s")
    run = pl.kernel(
        _sc_body,
        out_type=(half, half),
        mesh=mesh,
        compiler_params=pltpu.CompilerParams(use_tc_tiling_on_sc=False),
        scratch_types=[
            pltpu.VMEM_SHARED((NP, H), jnp.float32),
            pltpu.VMEM((NSUB * BG,), jnp.int32),
            pltpu.VMEM((NSUB * BG,), jnp.int32),
            pltpu.VMEM((NSUB, BG, H), jnp.float32),
            pltpu.SemaphoreType.DMA,
            pltpu.SemaphoreType.DMA,
        ],
    )
    return run(ylo, yhi, blo, bhi, src_blocks, dst_blocks)


@jax.jit
def kernel(features, edge_index, W_rel, b_rel, W_root):
    ylo, yhi, blo, bhi = _tc_stage(features, W_rel, b_rel, W_root)
    pad = E_PAD - E
    src = jnp.concatenate(
        [edge_index[0], jnp.zeros((pad,), jnp.int32)]
    ).reshape(BLOCKS, NSUB * BG)
    dst = jnp.concatenate(
        [edge_index[1], jnp.full((pad,), N, jnp.int32)]
    ).reshape(BLOCKS, NSUB * BG)
    out_lo, out_hi = _sc_stage(ylo, yhi, blo, bhi, src, dst)
    return jnp.concatenate([out_lo[:N], out_hi[:N]], axis=1)


# speedup vs baseline: 7.1746x; 1.0122x over previous
import jax
import jax.numpy as jnp
from jax import lax
from jax.experimental import pallas as pl
from jax.experimental.pallas import tpu as pltpu
from jax.experimental.pallas import tpu_sc as plsc

N = 50000
NP = 50048
E = 800000
D = 64
H = 32

NC = 2
NS = 16
BG = 128
NSUB = 3
CHUNKS = 132
EDGES_PER_TILE = CHUNKS * NSUB * BG
E_PAD = NS * EDGES_PER_TILE
ROWS_PER_TILE = NP // NS
BLOCKS = NS * CHUNKS

BM = ROWS_PER_TILE


def _tc_body(x_ref, wrel_ref, wroot_ref, b_ref,
             ylo_ref, yhi_ref, blo_ref, bhi_ref):
    x = x_ref[...]
    y = jnp.dot(x, wrel_ref[...], preferred_element_type=jnp.float32)
    base = jnp.dot(x, wroot_ref[...], preferred_element_type=jnp.float32)
    base = base + b_ref[...]
    ylo_ref[...] = y[:, :H]
    yhi_ref[...] = y[:, H:]
    blo_ref[...] = base[:, :H]
    bhi_ref[...] = base[:, H:]


def _tc_stage(features, W_rel, b_rel, W_root):
    grid = (NP // BM,)
    half = jax.ShapeDtypeStruct((NP, H), jnp.float32)
    return pl.pallas_call(
        _tc_body,
        grid=grid,
        in_specs=[
            pl.BlockSpec((BM, D), lambda i: (i, 0)),
            pl.BlockSpec((D, D), lambda i: (0, 0)),
            pl.BlockSpec((D, D), lambda i: (0, 0)),
            pl.BlockSpec((1, D), lambda i: (0, 0)),
        ],
        out_specs=[pl.BlockSpec((BM, H), lambda i: (i, 0))] * 4,
        out_shape=[half, half, half, half],
    )(features, W_rel, W_root, b_rel.reshape(1, D))


def _sc_body(ylo, yhi, blo, bhi, src_hbm, dst_hbm, out_lo, out_hi,
             acc, src_buf, dst_buf, rows, gsem, ssem):
    c = lax.axis_index("c")
    s = lax.axis_index("s")
    stripe = pl.ds(s * ROWS_PER_TILE, ROWS_PER_TILE)

    @pl.when(c == 0)
    def _():
        pltpu.sync_copy(blo.at[stripe], acc.at[stripe])

    @pl.when(c == 1)
    def _():
        pltpu.sync_copy(bhi.at[stripe], acc.at[stripe])

    plsc.subcore_barrier()

    def edge_loop(y_tab):
        def fire_gather(k, p):
            pltpu.sync_copy(src_hbm.at[s * CHUNKS + k], src_buf.at[p])
            pltpu.sync_copy(dst_hbm.at[s * CHUNKS + k], dst_buf.at[p])
            for j in range(NSUB):
                pltpu.async_copy(
                    y_tab.at[src_buf.at[p, pl.ds(j * BG, BG)]],
                    rows.at[p, j], gsem[p])

        def wait_gather(p):
            for j in range(NSUB):
                pltpu.make_async_copy(
                    y_tab.at[src_buf.at[p, pl.ds(j * BG, BG)]],
                    rows.at[p, j], gsem[p]).wait()

        def fire_scatter(p):
            for j in range(NSUB):
                pltpu.async_copy(
                    rows.at[p, j],
                    acc.at[dst_buf.at[p, pl.ds(j * BG, BG)]],
                    ssem[p], add=True)

        def wait_scatter(p):
            for j in range(NSUB):
                pltpu.make_async_copy(
                    rows.at[p, j],
                    acc.at[dst_buf.at[p, pl.ds(j * BG, BG)]],
                    ssem[p]).wait()

        fire_gather(0, 0)
        fire_gather(1, 1)
        wait_gather(0)
        fire_scatter(0)

        def body(i, carry):
            k = 2 * i
            wait_scatter(0)
            fire_gather(k, 0)
            wait_gather(1)
            fire_scatter(1)
            wait_scatter(1)
            fire_gather(k + 1, 1)
            wait_gather(0)
            fire_scatter(0)
            return carry
        lax.fori_loop(1, CHUNKS // 2, body, 0)

        wait_gather(1)
        fire_scatter(1)
        wait_scatter(0)
        wait_scatter(1)

    @pl.when(c == 0)
    def _():
        edge_loop(ylo)

    @pl.when(c == 1)
    def _():
        edge_loop(yhi)

    plsc.subcore_barrier()

    @pl.when(c == 0)
    def _():
        pltpu.sync_copy(acc.at[stripe], out_lo.at[stripe])

    @pl.when(c == 1)
    def _():
        pltpu.sync_copy(acc.at[stripe], out_hi.at[stripe])


def _sc_stage(ylo, yhi, blo, bhi, src_blocks, dst_blocks):
    half = jax.ShapeDtypeStruct((NP, H), jnp.float32)
    mesh = plsc.VectorSubcoreMesh(core_axis_name="c", subcore_axis_name="s")
    run = pl.kernel(
        _sc_body,
        out_type=(half, half),
        mesh=mesh,
        compiler_params=pltpu.CompilerParams(use_tc_tiling_on_sc=False),
        scratch_types=[
            pltpu.VMEM_SHARED((NP, H), jnp.float32),
            pltpu.VMEM((2, NSUB * BG), jnp.int32),
            pltpu.VMEM((2, NSUB * BG), jnp.int32),
            pltpu.VMEM((2, NSUB, BG, H), jnp.float32),
            (pltpu.SemaphoreType.DMA, pltpu.SemaphoreType.DMA),
            (pltpu.SemaphoreType.DMA, pltpu.SemaphoreType.DMA),
        ],
    )
    return run(ylo, yhi, blo, bhi, src_blocks, dst_blocks)


@jax.jit
def kernel(features, edge_index, W_rel, b_rel, W_root):
    ylo, yhi, blo, bhi = _tc_stage(features, W_rel, b_rel, W_root)
    pad = E_PAD - E
    src = jnp.concatenate(
        [edge_index[0], jnp.zeros((pad,), jnp.int32)]
    ).reshape(BLOCKS, NSUB * BG)
    dst = jnp.concatenate(
        [edge_index[1], jnp.full((pad,), N, jnp.int32)]
    ).reshape(BLOCKS, NSUB * BG)
    out_lo, out_hi = _sc_stage(ylo, yhi, blo, bhi, src, dst)
    return jnp.concatenate([out_lo[:N], out_hi[:N]], axis=1)
